# baseline (speedup 1.0000x reference)
import functools

import numpy as np

import jax
import jax.numpy as jnp
from jax.experimental import pallas as pl
from jax.experimental.pallas import tpu as pltpu

_BLOCK_W = 2048
_SEED = 8


def _rot_consts(D, n=10000.0):
    i = np.arange(D // 2, dtype=np.float64)
    ang = _SEED * np.power(n, -2.0 * i / D)
    c = np.repeat(np.cos(ang), 2)
    s = np.repeat(np.sin(ang), 2)
    return np.stack([c, s]).astype(np.float32)


def _add_kernel(n_steps, x_ref, seed_ref, cs_ref, o_ref, p_ref):
    i = pl.program_id(0)
    j = pl.program_id(1)

    @pl.when(j == 0)
    def _():
        c8 = cs_ref[0:1, :]
        s8 = cs_ref[1:2, :]
        c16 = c8 * c8 - s8 * s8
        s16 = 2.0 * c8 * s8
        qa = seed_ref[i, 0]
        ra = seed_ref[i, 1]
        qb = qa * c8 + ra * s8
        rb = ra * c8 - qa * s8
        p_ref[0:_SEED, :] = qa
        p_ref[_SEED : 2 * _SEED, :] = qb

        def step(k, carry):
            qa, ra, qb, rb = carry
            qa2 = qa * c16 + ra * s16
            ra2 = ra * c16 - qa * s16
            qb2 = qb * c16 + rb * s16
            rb2 = rb * c16 - qb * s16
            p_ref[pl.ds(k * 2 * _SEED, _SEED), :] = qa2
            p_ref[pl.ds(k * 2 * _SEED + _SEED, _SEED), :] = qb2
            return qa2, ra2, qb2, rb2

        jax.lax.fori_loop(1, n_steps, step, (qa, ra, qb, rb), unroll=4)

    o_ref[0] = x_ref[0] + p_ref[...]


def kernel(x, P):
    B, W, D = x.shape
    n_blocks = W // _BLOCK_W
    n_steps = _BLOCK_W // (2 * _SEED)
    q0 = P.reshape(n_blocks, _BLOCK_W, D)[:, :_SEED, :]
    qp = q0.reshape(n_blocks, _SEED, D // 2, 2)
    r0 = jnp.stack([qp[..., 1], -qp[..., 0]], axis=-1).reshape(q0.shape)
    seeds = jnp.stack([q0, r0], axis=1)
    cs = jnp.asarray(_rot_consts(D))

    grid = (n_blocks, B)
    return pl.pallas_call(
        functools.partial(_add_kernel, n_steps),
        grid=grid,
        in_specs=[
            pl.BlockSpec((1, _BLOCK_W, D), lambda i, j: (j, i, 0)),
            pl.BlockSpec((n_blocks, 2, _SEED, D), lambda i, j: (0, 0, 0, 0)),
            pl.BlockSpec((2, D), lambda i, j: (0, 0)),
        ],
        out_specs=pl.BlockSpec((1, _BLOCK_W, D), lambda i, j: (j, i, 0)),
        out_shape=jax.ShapeDtypeStruct((B, W, D), x.dtype),
        scratch_shapes=[pltpu.VMEM((_BLOCK_W, D), jnp.float32)],
        compiler_params=pltpu.CompilerParams(
            dimension_semantics=("arbitrary", "arbitrary"),
        ),
    )(x, seeds, cs)


# speedup vs baseline: 1.1027x; 1.1027x over previous
import jax
import jax.numpy as jnp
from jax.experimental import pallas as pl
from jax.experimental.pallas import tpu as pltpu


def _add_kernel(xa_ref, xb_ref, pa_ref, pb_ref, o_ref):
    half = xa_ref.shape[2]
    o_ref[0, :half] = xa_ref[0, 0] + pa_ref[0, 0]
    o_ref[0, half:] = xb_ref[0, 0] + pb_ref[0, 0]


def kernel(x, P):
    B, W, D = x.shape
    half = W // 2
    x4 = x.reshape(B, 2, half, D)
    P4 = P.reshape(1, 2, half, D)
    return pl.pallas_call(
        _add_kernel,
        grid=(B,),
        in_specs=[
            pl.BlockSpec((1, 1, half, D), lambda j: (j, 0, 0, 0)),
            pl.BlockSpec((1, 1, half, D), lambda j: (j, 1, 0, 0)),
            pl.BlockSpec((1, 1, half, D), lambda j: (0, 0, 0, 0)),
            pl.BlockSpec((1, 1, half, D), lambda j: (0, 1, 0, 0)),
        ],
        out_specs=pl.BlockSpec((1, W, D), lambda j: (j, 0, 0)),
        out_shape=jax.ShapeDtypeStruct((B, W, D), x.dtype),
        compiler_params=pltpu.CompilerParams(
            dimension_semantics=("arbitrary",),
        ),
    )(x4, x4, P4, P4)
